# baseline (speedup 1.0000x reference)
import jax
import jax.numpy as jnp
from jax import lax
from jax.experimental import pallas as pl
from jax.experimental.pallas import tpu as pltpu

_K = 8
_EPS = 1e-5


def _fused(x_ref, c_ref, g_ref, b_ref, o_ref,
           s1_scr, s2_scr, d_scr, sc_scr, sh_scr):
    i = pl.program_id(0)
    nb = pl.num_programs(0) // 2
    bb = x_ref.shape[0]

    @pl.when(i < nb)
    def _phase0():
        xb = x_ref[...]
        cc = c_ref[...]
        s1 = jnp.sum(xb, axis=2)
        s2 = jnp.sum(xb * xb, axis=2)
        m2 = jnp.sum(s2, axis=1)
        c2 = jnp.sum(jnp.sum(cc * cc, axis=2), axis=1)
        cols = []
        for k in range(_K):
            u = jnp.sum(xb * cc[k], axis=1)
            pk = jnp.sum(u, axis=1)
            cols.append(pk[:, None])
        mc = jnp.concatenate(cols, axis=1)
        row = pl.ds(i * bb, bb)
        d_scr[row, :] = jnp.abs(m2[:, None] + c2[None, :] - 2.0 * mc)
        s1_scr[row, :] = s1
        s2_scr[row, :] = s2

    @pl.when(i == nb)
    def _mid():
        d = d_scr[...]
        B = d.shape[0]
        kio = lax.broadcasted_iota(jnp.int32, (B, _K), 1)
        dmin = jnp.min(d, axis=1, keepdims=True)
        assign = jnp.min(jnp.where(d == dmin, kio, _K), axis=1)
        onehot = (kio == assign[:, None]).astype(jnp.float32)
        cnt = jnp.sum(onehot, axis=0) * 196.0
        sums = lax.dot_general(onehot, s1_scr[...], (((0,), (0,)), ((), ())),
                               preferred_element_type=jnp.float32)
        sumsq = lax.dot_general(onehot, s2_scr[...], (((0,), (0,)), ((), ())),
                                preferred_element_type=jnp.float32)
        denom = jnp.maximum(cnt, 1.0)[:, None]
        mean = sums / denom
        var = sumsq / denom - mean * mean
        inv = lax.rsqrt(var + _EPS)
        scale = g_ref[...] * inv
        shift = b_ref[...] - mean * scale
        sc_scr[...] = lax.dot_general(onehot, scale, (((1,), (0,)), ((), ())),
                                      preferred_element_type=jnp.float32)
        sh_scr[...] = lax.dot_general(onehot, shift, (((1,), (0,)), ((), ())),
                                      preferred_element_type=jnp.float32)

    @pl.when(i >= nb)
    def _phase1():
        row = pl.ds((i - nb) * bb, bb)
        scb = sc_scr[row, :]
        shb = sh_scr[row, :]
        o_ref[...] = x_ref[...] * scb[:, :, None] + shb[:, :, None]


def kernel(x, c, gamma, beta):
    B, C, H, W = x.shape
    HW = H * W
    x3 = x.reshape(B, C, HW)
    c3 = c.reshape(_K, C, HW)
    bb = 8
    nb = B // bb

    def xmap(i):
        j = jnp.where(i < nb, i, i - nb)
        return (j, 0, 0)

    def omap(i):
        j = jnp.where(i < nb, 0, i - nb)
        return (j, 0, 0)

    out3 = pl.pallas_call(
        _fused,
        grid=(2 * nb,),
        in_specs=[
            pl.BlockSpec((bb, C, HW), xmap),
            pl.BlockSpec((_K, C, HW), lambda i: (0, 0, 0)),
            pl.BlockSpec((_K, C), lambda i: (0, 0)),
            pl.BlockSpec((_K, C), lambda i: (0, 0)),
        ],
        out_specs=pl.BlockSpec((bb, C, HW), omap),
        out_shape=jax.ShapeDtypeStruct((B, C, HW), jnp.float32),
        scratch_shapes=[
            pltpu.VMEM((B, C), jnp.float32),
            pltpu.VMEM((B, C), jnp.float32),
            pltpu.VMEM((B, _K), jnp.float32),
            pltpu.VMEM((B, C), jnp.float32),
            pltpu.VMEM((B, C), jnp.float32),
        ],
    )(x3, c3, gamma, beta)

    return out3.reshape(B, C, H, W)
